# Initial kernel scaffold; baseline (speedup 1.0000x reference)
import jax
import jax.numpy as jnp
from jax.experimental import pallas as pl


def kernel(pos, batch, params):
    raise NotImplementedError("write your pallas kernel here")


# speedup vs baseline: 4.1460x; 4.1460x over previous
import functools
import jax
import jax.numpy as jnp
from jax import lax
from jax.experimental import pallas as pl
from jax.experimental.pallas import tpu as pltpu
from jax.experimental.pallas import tpu_sc as plsc

NP = 10240
BQ = 256
GRID = NP // BQ
K = 16
NE = NP * K
BIG = 1e30
NEG = float("-inf")

_f32 = jnp.float32


def _dot(a, b):
    return jnp.dot(a, b, preferred_element_type=jnp.float32)


def _knn_body(posq_ref, posT_ref, bq_ref, bkT_ref, wx_ref, wp_ref, b1_ref,
              nbr_ref, y1_ref, t1_ref):
    d2 = jnp.zeros((BQ, NP), jnp.float32)
    for c in range(3):
        qc = posq_ref[:, c:c + 1]
        kc = posT_ref[c:c + 1, :]
        d = qc - kc
        d2 = d2 + d * d
    mask = bq_ref[...] != bkT_ref[...]
    d2 = jnp.where(mask, BIG, d2)
    iota = lax.broadcasted_iota(jnp.int32, (1, NP), 1)
    cols = []
    for _ in range(K):
        m = jnp.min(d2, axis=1, keepdims=True)
        cand = jnp.where(d2 == m, iota, NP)
        idx = jnp.min(cand, axis=1, keepdims=True)
        cols.append(idx)
        d2 = jnp.where(iota == idx, jnp.float32(jnp.inf), d2)
    nbr_ref[...] = jnp.concatenate(cols, axis=1)
    pq = posq_ref[...]
    pw = _dot(pq, wp_ref[...])
    y1_ref[...] = _dot(pq, wx_ref[...]) + pw
    t1_ref[...] = b1_ref[...] - pw


def _knn_call(posp, posT, bq, bkT, wx, wp, b1):
    return pl.pallas_call(
        _knn_body,
        grid=(GRID,),
        in_specs=[
            pl.BlockSpec((BQ, 3), lambda i: (i, 0)),
            pl.BlockSpec((3, NP), lambda i: (0, 0)),
            pl.BlockSpec((BQ, 1), lambda i: (i, 0)),
            pl.BlockSpec((1, NP), lambda i: (0, 0)),
            pl.BlockSpec((3, 128), lambda i: (0, 0)),
            pl.BlockSpec((3, 128), lambda i: (0, 0)),
            pl.BlockSpec((1, 128), lambda i: (0, 0)),
        ],
        out_specs=[
            pl.BlockSpec((BQ, K), lambda i: (i, 0)),
            pl.BlockSpec((BQ, 128), lambda i: (i, 0)),
            pl.BlockSpec((BQ, 128), lambda i: (i, 0)),
        ],
        out_shape=[
            jax.ShapeDtypeStruct((NP, K), jnp.int32),
            jax.ShapeDtypeStruct((NP, 128), jnp.float32),
            jax.ShapeDtypeStruct((NP, 128), jnp.float32),
        ],
    )(posp, posT, bq, bkT, wx, wp, b1)


def _sc_gather(table, idx):
    F = 128
    NW = 32
    bpw = NE // NW
    ch = 81920 // F
    nch = bpw // ch
    mesh = plsc.VectorSubcoreMesh(core_axis_name="c", subcore_axis_name="s")

    @functools.partial(
        pl.kernel, mesh=mesh,
        out_type=jax.ShapeDtypeStruct((NE, F), jnp.float32),
        scratch_types=[
            pltpu.VMEM((ch,), jnp.int32),
            pltpu.VMEM((ch, F), jnp.float32),
            pltpu.SemaphoreType.DMA,
        ],
    )
    def k(table_hbm, idx_hbm, out_hbm, idx_v, rows_v, sem):
        wid = lax.axis_index("s") * 2 + lax.axis_index("c")
        base = wid * bpw
        for c in range(nch):
            off = base + c * ch
            pltpu.sync_copy(idx_hbm.at[pl.ds(off, ch)], idx_v)
            pltpu.async_copy(table_hbm.at[idx_v], rows_v, sem).wait()
            pltpu.sync_copy(rows_v, out_hbm.at[pl.ds(off, ch)])

    return k(table, idx)


def _edge_body_next(Fb, yg_ref, t_ref, pos_ref, wb_ref, bb_ref,
                    wxn_ref, wpn_ref, bn_ref, h_ref, yn_ref, tn_ref):
    t = t_ref[...]
    wb = wb_ref[...]
    bb = bb_ref[...]
    m = jnp.full((BQ, Fb), NEG, jnp.float32)
    for j in range(K):
        pre = jnp.maximum(yg_ref[j] + t, 0.0)
        m = jnp.maximum(m, _dot(pre, wb) + bb)
    h = jnp.maximum(m, 0.0)
    h_ref[...] = h
    p = pos_ref[...]
    pw = _dot(p, wpn_ref[...])
    yn_ref[...] = _dot(h, wxn_ref[...]) + pw
    tn_ref[...] = bn_ref[...] - pw


def _edge_body_last(Fb, yg_ref, t_ref, wb_ref, bb_ref, h_ref):
    t = t_ref[...]
    wb = wb_ref[...]
    bb = bb_ref[...]
    m = jnp.full((BQ, Fb), NEG, jnp.float32)
    for j in range(K):
        pre = jnp.maximum(yg_ref[j] + t, 0.0)
        m = jnp.maximum(m, _dot(pre, wb) + bb)
    h_ref[...] = jnp.maximum(m, 0.0)


def _edge_call_next(yg3, t, posp, wb, bb, wxn, wpn, bn, Fb):
    return pl.pallas_call(
        functools.partial(_edge_body_next, Fb),
        grid=(GRID,),
        in_specs=[
            pl.BlockSpec((K, BQ, 128), lambda i: (0, i, 0)),
            pl.BlockSpec((BQ, 128), lambda i: (i, 0)),
            pl.BlockSpec((BQ, 3), lambda i: (i, 0)),
            pl.BlockSpec((128, Fb), lambda i: (0, 0)),
            pl.BlockSpec((1, Fb), lambda i: (0, 0)),
            pl.BlockSpec((Fb, 128), lambda i: (0, 0)),
            pl.BlockSpec((3, 128), lambda i: (0, 0)),
            pl.BlockSpec((1, 128), lambda i: (0, 0)),
        ],
        out_specs=[
            pl.BlockSpec((BQ, Fb), lambda i: (i, 0)),
            pl.BlockSpec((BQ, 128), lambda i: (i, 0)),
            pl.BlockSpec((BQ, 128), lambda i: (i, 0)),
        ],
        out_shape=[
            jax.ShapeDtypeStruct((NP, Fb), jnp.float32),
            jax.ShapeDtypeStruct((NP, 128), jnp.float32),
            jax.ShapeDtypeStruct((NP, 128), jnp.float32),
        ],
    )(yg3, t, posp, wb, bb, wxn, wpn, bn)


def _edge_call_last(yg3, t, wb, bb, Fb):
    return pl.pallas_call(
        functools.partial(_edge_body_last, Fb),
        grid=(GRID,),
        in_specs=[
            pl.BlockSpec((K, BQ, 128), lambda i: (0, i, 0)),
            pl.BlockSpec((BQ, 128), lambda i: (i, 0)),
            pl.BlockSpec((128, Fb), lambda i: (0, 0)),
            pl.BlockSpec((1, Fb), lambda i: (0, 0)),
        ],
        out_specs=pl.BlockSpec((BQ, Fb), lambda i: (i, 0)),
        out_shape=jax.ShapeDtypeStruct((NP, Fb), jnp.float32),
    )(yg3, t, wb, bb)


def _seg_max_update(out_ref, w, bq):
    i = pl.program_id(0)

    @pl.when(i == 0)
    def _():
        out_ref[...] = jnp.full(out_ref.shape, NEG, jnp.float32)

    parts = []
    for g in range(8):
        mg = jnp.max(jnp.where(bq == _f32(g), w, NEG), axis=0, keepdims=True)
        parts.append(mg)
    out_ref[...] = jnp.maximum(out_ref[...], jnp.concatenate(parts, axis=0))


def _stageA_body(des_ref, bq_ref, w1_ref, b1_ref, w2_ref, b2_ref,
                 w3_ref, b3_ref, out_ref):
    x = des_ref[...]
    u = jnp.maximum(_dot(x, w1_ref[...]) + b1_ref[...], 0.0)
    u = jnp.maximum(_dot(u, w2_ref[...]) + b2_ref[...], 0.0)
    w = _dot(u, w3_ref[...]) + b3_ref[...]
    _seg_max_update(out_ref, w, bq_ref[...])


def _stageA_call(des, bq, w1, b1, w2, b2, w3, b3):
    return pl.pallas_call(
        _stageA_body,
        grid=(GRID,),
        in_specs=[
            pl.BlockSpec((BQ, 224), lambda i: (i, 0)),
            pl.BlockSpec((BQ, 1), lambda i: (i, 0)),
            pl.BlockSpec((224, 256), lambda i: (0, 0)),
            pl.BlockSpec((1, 256), lambda i: (0, 0)),
            pl.BlockSpec((256, 512), lambda i: (0, 0)),
            pl.BlockSpec((1, 512), lambda i: (0, 0)),
            pl.BlockSpec((512, 512), lambda i: (0, 0)),
            pl.BlockSpec((1, 512), lambda i: (0, 0)),
        ],
        out_specs=pl.BlockSpec((8, 512), lambda i: (0, 0)),
        out_shape=jax.ShapeDtypeStruct((8, 512), jnp.float32),
    )(des, bq, w1, b1, w2, b2, w3, b3)


def _onehot8(bq):
    i8 = lax.broadcasted_iota(jnp.int32, (1, 8), 1).astype(jnp.float32)
    return (bq == i8).astype(jnp.float32)


def _stageB_body(des_ref, bq_ref, gp_ref, wa_ref, wg_ref, ba_ref,
                 w2_ref, b2_ref, out_ref):
    bq = bq_ref[...]
    gb = _dot(_onehot8(bq), gp_ref[...])
    u = jnp.maximum(_dot(des_ref[...], wa_ref[...]) + _dot(gb, wg_ref[...])
                    + ba_ref[...], 0.0)
    v = _dot(u, w2_ref[...]) + b2_ref[...]
    _seg_max_update(out_ref, v, bq)


def _stageB_call(des, bq, gpool, wa, wg, ba, w2, b2):
    return pl.pallas_call(
        _stageB_body,
        grid=(GRID,),
        in_specs=[
            pl.BlockSpec((BQ, 224), lambda i: (i, 0)),
            pl.BlockSpec((BQ, 1), lambda i: (i, 0)),
            pl.BlockSpec((8, 512), lambda i: (0, 0)),
            pl.BlockSpec((224, 1024), lambda i: (0, 0)),
            pl.BlockSpec((512, 1024), lambda i: (0, 0)),
            pl.BlockSpec((1, 1024), lambda i: (0, 0)),
            pl.BlockSpec((1024, 1024), lambda i: (0, 0)),
            pl.BlockSpec((1, 1024), lambda i: (0, 0)),
        ],
        out_specs=pl.BlockSpec((8, 1024), lambda i: (0, 0)),
        out_shape=jax.ShapeDtypeStruct((8, 1024), jnp.float32),
    )(des, bq, gpool, wa, wg, ba, w2, b2)


def _stageC_body(des_ref, bq_ref, ge_ref, wa_ref, wg_ref, ba_ref,
                 w2_ref, b2_ref, w3_ref, b3_ref, w4_ref, b4_ref, out_ref):
    gb = _dot(_onehot8(bq_ref[...]), ge_ref[...])
    u = jnp.maximum(_dot(des_ref[...], wa_ref[...]) + _dot(gb, wg_ref[...])
                    + ba_ref[...], 0.0)
    u = jnp.maximum(_dot(u, w2_ref[...]) + b2_ref[...], 0.0)
    u = jnp.maximum(_dot(u, w3_ref[...]) + b3_ref[...], 0.0)
    o = _dot(u, w4_ref[...]) + b4_ref[...]
    out_ref[...] = jnp.broadcast_to(o, (BQ, 128))


def _stageC_call(des, bq, gemb, wa, wg, ba, w2, b2, w3, b3, w4, b4):
    return pl.pallas_call(
        _stageC_body,
        grid=(GRID,),
        in_specs=[
            pl.BlockSpec((BQ, 224), lambda i: (i, 0)),
            pl.BlockSpec((BQ, 1), lambda i: (i, 0)),
            pl.BlockSpec((8, 1024), lambda i: (0, 0)),
            pl.BlockSpec((224, 512), lambda i: (0, 0)),
            pl.BlockSpec((1024, 512), lambda i: (0, 0)),
            pl.BlockSpec((1, 512), lambda i: (0, 0)),
            pl.BlockSpec((512, 256), lambda i: (0, 0)),
            pl.BlockSpec((1, 256), lambda i: (0, 0)),
            pl.BlockSpec((256, 128), lambda i: (0, 0)),
            pl.BlockSpec((1, 128), lambda i: (0, 0)),
            pl.BlockSpec((128, 1), lambda i: (0, 0)),
            pl.BlockSpec((1, 1), lambda i: (0, 0)),
        ],
        out_specs=pl.BlockSpec((BQ, 128), lambda i: (i, 0)),
        out_shape=jax.ShapeDtypeStruct((NP, 128), jnp.float32),
    )(des, bq, gemb, wa, wg, ba, w2, b2, w3, b3, w4, b4)


def kernel(pos, batch, params):
    p = params
    N = pos.shape[0]
    posp = jnp.zeros((NP, 3), jnp.float32).at[:N].set(pos)
    bf = jnp.full((NP,), -1.0, jnp.float32).at[:N].set(batch.astype(jnp.float32))
    posT = posp.T
    bq = bf[:, None]
    bkT = bf[None, :]

    def padc(w):
        return jnp.pad(w, ((0, 0), (0, 128 - w.shape[1])))

    def padr(w):
        return jnp.pad(w, ((0, 128 - w.shape[0]), (0, 0)))

    w1a = p["c1a"]["w"]
    nbr, y1, t1 = _knn_call(posp, posT, bq, bkT,
                            padc(w1a[:3]), padc(w1a[3:]),
                            padc(p["c1a"]["b"][None]))
    idx = nbr.T.reshape(-1)

    w2a, w3a = p["c2a"]["w"], p["c3a"]["w"]
    yg1 = _sc_gather(y1, idx).reshape(K, NP, 128)
    h1, y2, t2 = _edge_call_next(yg1, t1, posp,
                                 padr(p["c1b"]["w"]), p["c1b"]["b"][None],
                                 padc(w2a[:32]), padc(w2a[32:]),
                                 padc(p["c2a"]["b"][None]), 32)
    yg2 = _sc_gather(y2, idx).reshape(K, NP, 128)
    h2, y3, t3 = _edge_call_next(yg2, t2, posp,
                                 padr(p["c2b"]["w"]), p["c2b"]["b"][None],
                                 w3a[:64], w3a[64:], p["c3a"]["b"][None], 64)
    yg3 = _sc_gather(y3, idx).reshape(K, NP, 128)
    h3 = _edge_call_last(yg3, t3, p["c3b"]["w"], p["c3b"]["b"][None], 128)

    des = jnp.concatenate([h1, h2, h3], axis=1)

    gpool = _stageA_call(des, bq,
                         p["g1a"]["w"], p["g1a"]["b"][None],
                         p["g1b"]["w"], p["g1b"]["b"][None],
                         p["g1c"]["w"], p["g1c"]["b"][None])
    g2w = p["g2a"]["w"]
    gemb = _stageB_call(des, bq, gpool,
                        g2w[:224], g2w[224:], p["g2a"]["b"][None],
                        p["g2b"]["w"], p["g2b"]["b"][None])
    c1w = p["cls1"]["w"]
    out = _stageC_call(des, bq, gemb,
                       c1w[:224], c1w[224:], p["cls1"]["b"][None],
                       p["cls2"]["w"], p["cls2"]["b"][None],
                       p["cls3"]["w"], p["cls3"]["b"][None],
                       p["cls4"]["w"], p["cls4"]["b"][None])
    return out[:N, 0:1]
